# baseline (speedup 1.0000x reference)
import functools

import jax
import jax.numpy as jnp
from jax import lax
from jax.experimental import pallas as pl
from jax.experimental.pallas import tpu as pltpu
from jax.experimental.pallas import tpu_sc as plsc

BATCH = 4096
SEQ = 200
EMBED = 64
NUM_WORKERS = 32
SEQ_PER_WORKER = BATCH // NUM_WORKERS
_CHUNKS = ((0, 104), (104, 96))


def _encoder_body(ids_hbm, pos_hbm, table_hbm, out_hbm, idx_v, rows_v, pos_v, sem):
    wid = lax.axis_index("s") * 2 + lax.axis_index("c")
    pltpu.sync_copy(pos_hbm, pos_v)

    def seq_body(i, carry):
        seq = wid * SEQ_PER_WORKER + i
        pltpu.sync_copy(ids_hbm.at[seq], idx_v)
        copies = [
            pltpu.async_copy(
                table_hbm.at[idx_v.at[pl.ds(off, n)]],
                rows_v.at[pl.ds(off, n)],
                sem,
            )
            for off, n in _CHUNKS
        ]
        for cp in copies:
            cp.wait()

        def row_body(l, c):
            for k in range(EMBED // 16):
                sl = pl.ds(k * 16, 16)
                rows_v[l, sl] = rows_v[l, sl] + pos_v[l, sl]
            return c

        lax.fori_loop(0, SEQ, row_body, 0)
        pltpu.sync_copy(rows_v, out_hbm.at[pl.ds(seq * SEQ, SEQ)])
        return carry

    lax.fori_loop(0, SEQ_PER_WORKER, seq_body, 0)


def kernel(input_ids, token_table, pos_embedding):
    pos2d = pos_embedding.reshape(SEQ, EMBED)
    mesh = plsc.VectorSubcoreMesh(core_axis_name="c", subcore_axis_name="s")
    run = pl.kernel(
        _encoder_body,
        mesh=mesh,
        out_type=jax.ShapeDtypeStruct((BATCH * SEQ, EMBED), jnp.float32),
        scratch_types=[
            pltpu.VMEM((SEQ,), jnp.int32),
            pltpu.VMEM((SEQ, EMBED), jnp.float32),
            pltpu.VMEM((SEQ, EMBED), jnp.float32),
            pltpu.SemaphoreType.DMA,
        ],
        compiler_params=pltpu.CompilerParams(use_tc_tiling_on_sc=False),
    )
    out = run(input_ids, pos2d, token_table)
    return out.reshape(BATCH, SEQ, EMBED)


# speedup vs baseline: 1.1923x; 1.1923x over previous
import jax
import jax.numpy as jnp
from jax import lax
from jax.experimental import pallas as pl
from jax.experimental.pallas import tpu as pltpu
from jax.experimental.pallas import tpu_sc as plsc

BATCH = 4096
SEQ = 200
EMBED = 64
NUM_WORKERS = 32
SEQ_PER_WORKER = BATCH // NUM_WORKERS
NBUF = 4
NGROUPS = SEQ_PER_WORKER // NBUF
_CHUNKS = ((0, 104), (104, 96))


def _encoder_body(ids_hbm, pos_hbm, table_hbm, out_hbm, idx_all, pos_v, *bufs):
    rows = bufs[:NBUF]
    sem_g = bufs[NBUF:2 * NBUF]
    sem_w = bufs[2 * NBUF:3 * NBUF]
    wid = lax.axis_index("s") * 2 + lax.axis_index("c")
    seq0 = wid * SEQ_PER_WORKER
    pltpu.sync_copy(pos_hbm, pos_v)
    pltpu.sync_copy(ids_hbm.at[pl.ds(seq0, SEQ_PER_WORKER)], idx_all)

    def start_gather(t, b):
        for off, n in _CHUNKS:
            pltpu.async_copy(
                table_hbm.at[idx_all.at[t, pl.ds(off, n)]],
                rows[b].at[pl.ds(off, n)],
                sem_g[b],
            )

    def wait_gather(b):
        for off, n in _CHUNKS:
            pltpu.make_async_copy(
                table_hbm.at[idx_all.at[0, pl.ds(off, n)]],
                rows[b].at[pl.ds(off, n)],
                sem_g[b],
            ).wait()

    def wait_wb(b):
        pltpu.make_async_copy(
            rows[b], out_hbm.at[pl.ds(0, SEQ)], sem_w[b]
        ).wait()

    start_gather(0, 0)
    start_gather(1, 1)

    def group_body(g, carry):
        for b in range(NBUF):
            t = g * NBUF + b

            wait_gather(b)

            def row_body(i, c):
                for du in range(2):
                    l = 2 * i + du
                    for k in range(EMBED // 16):
                        sl = pl.ds(k * 16, 16)
                        rows[b][l, sl] = rows[b][l, sl] + pos_v[l, sl]
                return c

            lax.fori_loop(0, SEQ // 2, row_body, 0)

            pltpu.async_copy(
                rows[b], out_hbm.at[pl.ds((seq0 + t) * SEQ, SEQ)], sem_w[b]
            )

            b2 = (b + 2) % NBUF
            if b < 2:
                @pl.when(g > 0)
                def _():
                    wait_wb(b2)

                start_gather(t + 2, b2)
            else:
                wait_wb(b2)

                @pl.when(g < NGROUPS - 1)
                def _():
                    start_gather(t + 2, b2)
        return carry

    lax.fori_loop(0, NGROUPS, group_body, 0)
    wait_wb(2)
    wait_wb(3)


def kernel(input_ids, token_table, pos_embedding):
    pos2d = pos_embedding.reshape(SEQ, EMBED)
    mesh = plsc.VectorSubcoreMesh(core_axis_name="c", subcore_axis_name="s")
    run = pl.kernel(
        _encoder_body,
        mesh=mesh,
        out_type=jax.ShapeDtypeStruct((BATCH * SEQ, EMBED), jnp.float32),
        scratch_types=[
            pltpu.VMEM((SEQ_PER_WORKER, SEQ), jnp.int32),
            pltpu.VMEM((SEQ, EMBED), jnp.float32),
        ]
        + [pltpu.VMEM((SEQ, EMBED), jnp.float32) for _ in range(NBUF)]
        + [pltpu.SemaphoreType.DMA for _ in range(2 * NBUF)],
        compiler_params=pltpu.CompilerParams(use_tc_tiling_on_sc=False),
    )
    out = run(input_ids, pos2d, token_table)
    return out.reshape(BATCH, SEQ, EMBED)
